# baseline (speedup 1.0000x reference)
import functools

import jax
import jax.numpy as jnp
import numpy as np
from jax import lax
from jax.experimental import pallas as pl
from jax.experimental.pallas import tpu as pltpu, tpu_sc as plsc

_B = 4096
_S = 32
_F = 128
_NW = 32
_BW = _B // _NW
_RPR = 4
_NR = _BW // _RPR
_NBUF = 4
_L = 16


def _sc_body(feat_hbm, fbf_hbm, nodes_hbm, nidx_hbm,
             self_out, nsum_out,
             idx_s, nodes_v, res_buf, self_buf,
             bufs, gsem, selfsem):
    c = lax.axis_index("c")
    q = lax.axis_index("s")
    w = c * 16 + q
    base = w * _BW

    pltpu.sync_copy(nidx_hbm.at[w], idx_s)

    pltpu.sync_copy(nodes_hbm.at[pl.ds(base, _BW)], nodes_v)
    pltpu.async_copy(feat_hbm.at[nodes_v], self_buf, selfsem)

    for b in range(_NBUF):
        pltpu.async_copy(fbf_hbm.at[idx_s.at[b]], bufs[b], gsem[b])

    def wait_gather(b):
        pltpu.make_async_copy(fbf_hbm.at[pl.ds(0, _RPR * _S)], bufs[b],
                              gsem[b]).wait()

    zero2 = jnp.zeros((2, _L), jnp.bfloat16)

    def consume(j, b):
        for r4 in range(_RPR):
            def add_pair(r, acc):
                row0 = pl.multiple_of(r4 * _S + 4 * r, 2)
                row1 = pl.multiple_of(r4 * _S + 4 * r + 2, 2)
                new = []
                for k in range(_F // _L):
                    v0 = bufs[b][pl.ds(row0, 2), pl.ds(k * _L, _L)]
                    v1 = bufs[b][pl.ds(row1, 2), pl.ds(k * _L, _L)]
                    new.append(acc[k] + (v0 + v1))
                return tuple(new)

            acc = lax.fori_loop(0, _S // 4, add_pair,
                                (zero2,) * (_F // _L))
            brow = j * _RPR + r4
            for k in range(_F // _L):
                af = acc[k].astype(jnp.float32)
                s0 = lax.squeeze(lax.slice(af, (0, 0), (1, _L)), (0,))
                s1 = lax.squeeze(lax.slice(af, (1, 0), (2, _L)), (0,))
                res_buf[brow, pl.ds(2 * k * _L, _L)] = s0
                res_buf[brow, pl.ds((2 * k + 1) * _L, _L)] = s1

    def group(g, carry):
        for b in range(_NBUF):
            j = g * _NBUF + b
            wait_gather(b)
            consume(j, b)
            pltpu.async_copy(fbf_hbm.at[idx_s.at[j + _NBUF]], bufs[b],
                             gsem[b])
        return carry

    lax.fori_loop(0, _NR // _NBUF - 1, group, 0)

    for j in range(_NR - _NBUF, _NR):
        b = j % _NBUF
        wait_gather(b)
        consume(j, b)

    pltpu.sync_copy(res_buf, nsum_out.at[pl.ds(base, _BW)])
    pltpu.make_async_copy(feat_hbm.at[pl.ds(0, _BW)], self_buf,
                          selfsem).wait()
    pltpu.sync_copy(self_buf, self_out.at[pl.ds(base, _BW)])


def _sc_gather(features, fbf, nodes, nidx):
    mesh = plsc.VectorSubcoreMesh(core_axis_name="c", subcore_axis_name="s")
    f32 = jnp.float32
    return pl.kernel(
        _sc_body,
        out_type=(jax.ShapeDtypeStruct((_B, _F), f32),
                  jax.ShapeDtypeStruct((_B, 2 * _F), f32)),
        mesh=mesh,
        compiler_params=pltpu.CompilerParams(use_tc_tiling_on_sc=False),
        scratch_types=[
            pltpu.VMEM((_NR, _RPR * _S), jnp.int32),
            pltpu.VMEM((_BW,), jnp.int32),
            pltpu.VMEM((_BW, 2 * _F), f32),
            pltpu.VMEM((_BW, _F), f32),
            [pltpu.VMEM((_RPR * _S, _F), jnp.bfloat16)
             for _ in range(_NBUF)],
            [pltpu.SemaphoreType.DMA for _ in range(_NBUF)],
            pltpu.SemaphoreType.DMA,
        ],
    )(features, fbf, nodes, nidx)


def _tc_body(self_ref, nsum_ref, w_ref, out_ref):
    w1 = w_ref[0:_F, :]
    w2d = w_ref[_F:3 * _F, :] * (1.0 / _S)
    a = lax.dot_general(w1, self_ref[...], (((0,), (1,)), ((), ())),
                        preferred_element_type=jnp.float32)
    b = lax.dot_general(w2d, nsum_ref[...], (((0,), (1,)), ((), ())),
                        preferred_element_type=jnp.float32)
    out_ref[...] = jnp.maximum(a + b, 0.0)


def _tc_project(self_feats, nsum, wmod):
    blk = 1024
    grid = (_B // blk,)
    return pl.pallas_call(
        _tc_body,
        grid=grid,
        in_specs=[
            pl.BlockSpec((blk, _F), lambda i: (i, 0)),
            pl.BlockSpec((blk, 2 * _F), lambda i: (i, 0)),
            pl.BlockSpec((3 * _F, _F), lambda i: (0, 0)),
        ],
        out_specs=pl.BlockSpec((_F, blk), lambda i: (0, i)),
        out_shape=jax.ShapeDtypeStruct((_F, _B), jnp.float32),
    )(self_feats, nsum, wmod)


_DUP_ORDER = np.array(
    [16 * (p // 32) + (p % 16) for p in range(2 * _F)], dtype=np.int32)


@jax.jit
def kernel(nodes, neigh_idx, features, weight):
    nodes = nodes.astype(jnp.int32)
    nidx = neigh_idx.astype(jnp.int32).reshape(_NW, _NR, _RPR * _S)
    fbf = features.astype(jnp.bfloat16)
    wmod = jnp.concatenate(
        [weight[:_F], weight[_F:][jnp.asarray(_DUP_ORDER)]], axis=0)
    self_feats, nsum = _sc_gather(features, fbf, nodes, nidx)
    return _tc_project(self_feats, nsum, wmod)


# speedup vs baseline: 1.2543x; 1.2543x over previous
import functools

import jax
import jax.numpy as jnp
from jax import lax
from jax.experimental import pallas as pl
from jax.experimental.pallas import tpu as pltpu, tpu_sc as plsc

_B = 4096
_S = 32
_F = 128
_NW = 32
_BW = _B // _NW
_NBUF = 8


def _sc_body(feat_hbm, fbf_hbm, nodes_hbm, neighT_hbm, loc_hbm,
             self_out, neigh_out,
             idx_s, nodes_v, loc_v, self_buf, acc_sh,
             bufs, gsem, ssem, selfsem):
    c = lax.axis_index("c")
    q = lax.axis_index("s")
    w = c * 16 + q
    base = w * _BW
    lbase = q * _BW

    pltpu.sync_copy(neighT_hbm.at[w], idx_s)
    pltpu.sync_copy(loc_hbm.at[pl.ds(base, _BW)], loc_v)

    pltpu.sync_copy(nodes_hbm.at[pl.ds(base, _BW)], nodes_v)
    pltpu.async_copy(feat_hbm.at[nodes_v], self_buf, selfsem)

    for b in range(_NBUF):
        pltpu.async_copy(fbf_hbm.at[idx_s.at[b]], bufs[b], gsem[b])

    def wait_gather(b):
        pltpu.make_async_copy(fbf_hbm.at[pl.ds(0, _BW)], bufs[b],
                              gsem[b]).wait()

    def wait_scatter(b):
        pltpu.make_async_copy(bufs[b], acc_sh.at[pl.ds(lbase, _BW)],
                              ssem[b]).wait()

    def do_round(j, b, refill_j):
        wait_gather(b)
        pltpu.async_copy(bufs[b], acc_sh.at[loc_v], ssem[b], add=True)
        if refill_j is not None:
            wait_scatter(b)
            pltpu.async_copy(fbf_hbm.at[idx_s.at[refill_j]], bufs[b],
                             gsem[b])

    wait_gather(0)
    pltpu.sync_copy(bufs[0], acc_sh.at[loc_v])
    pltpu.async_copy(fbf_hbm.at[idx_s.at[_NBUF]], bufs[0], gsem[0])

    for j in range(1, _NBUF):
        do_round(j, j, j + _NBUF)

    def group(g, carry):
        for b in range(_NBUF):
            j = g * _NBUF + b
            do_round(j, b, j + _NBUF)
        return carry

    lax.fori_loop(1, _S // _NBUF - 1, group, 0)

    for j in range(_S - _NBUF, _S):
        do_round(j, j % _NBUF, None)
    for b in range(_NBUF):
        wait_scatter(b)

    pltpu.make_async_copy(feat_hbm.at[pl.ds(0, _BW)], self_buf,
                          selfsem).wait()
    pltpu.sync_copy(self_buf, self_out.at[pl.ds(base, _BW)])
    pltpu.sync_copy(acc_sh.at[pl.ds(lbase, _BW)],
                    neigh_out.at[pl.ds(base, _BW)])


def _sc_gather(features, fbf, nodes, neighTw, loc):
    mesh = plsc.VectorSubcoreMesh(core_axis_name="c", subcore_axis_name="s")
    f32 = jnp.float32
    bf16 = jnp.bfloat16
    return pl.kernel(
        _sc_body,
        out_type=(jax.ShapeDtypeStruct((_B, _F), f32),
                  jax.ShapeDtypeStruct((_B, _F), bf16)),
        mesh=mesh,
        compiler_params=pltpu.CompilerParams(use_tc_tiling_on_sc=False),
        scratch_types=[
            pltpu.VMEM((_S, _BW), jnp.int32),
            pltpu.VMEM((_BW,), jnp.int32),
            pltpu.VMEM((_BW,), jnp.int32),
            pltpu.VMEM((_BW, _F), f32),
            pltpu.VMEM_SHARED((_B // 2, _F), bf16),
            [pltpu.VMEM((_BW, _F), bf16) for _ in range(_NBUF)],
            [pltpu.SemaphoreType.DMA for _ in range(_NBUF)],
            [pltpu.SemaphoreType.DMA for _ in range(_NBUF)],
            pltpu.SemaphoreType.DMA,
        ],
    )(features, fbf, nodes, neighTw, loc)


def _tc_body(self_ref, neigh_ref, w_ref, out_ref):
    w1 = w_ref[0:_F, :]
    w2 = w_ref[_F:2 * _F, :] * (1.0 / _S)
    a = lax.dot_general(w1, self_ref[...], (((0,), (1,)), ((), ())),
                        preferred_element_type=jnp.float32)
    b = lax.dot_general(w2, neigh_ref[...].astype(jnp.float32),
                        (((0,), (1,)), ((), ())),
                        preferred_element_type=jnp.float32)
    out_ref[...] = jnp.maximum(a + b, 0.0)


def _tc_project(self_feats, neigh_sum, weight):
    blk = 1024
    grid = (_B // blk,)
    return pl.pallas_call(
        _tc_body,
        grid=grid,
        in_specs=[
            pl.BlockSpec((blk, _F), lambda i: (i, 0)),
            pl.BlockSpec((blk, _F), lambda i: (i, 0)),
            pl.BlockSpec((2 * _F, _F), lambda i: (0, 0)),
        ],
        out_specs=pl.BlockSpec((_F, blk), lambda i: (0, i)),
        out_shape=jax.ShapeDtypeStruct((_F, _B), jnp.float32),
    )(self_feats, neigh_sum, weight)


@jax.jit
def kernel(nodes, neigh_idx, features, weight):
    nodes = nodes.astype(jnp.int32)
    neighTw = jnp.transpose(
        neigh_idx.astype(jnp.int32).reshape(_NW, _BW, _S), (0, 2, 1))
    loc = jnp.arange(_B, dtype=jnp.int32) % (_B // 2)
    fbf = features.astype(jnp.bfloat16)
    self_feats, neigh_sum = _sc_gather(features, fbf, nodes, neighTw, loc)
    return _tc_project(self_feats, neigh_sum, weight)
